# Initial kernel scaffold; baseline (speedup 1.0000x reference)
import jax
import jax.numpy as jnp
from jax.experimental import pallas as pl


def kernel(features, hidden_gru, hidden_states_h, hidden_states_c, degrees, norm, edge_index, params):
    raise NotImplementedError("write your pallas kernel here")


# speedup vs baseline: 1.1181x; 1.1181x over previous
import functools

import jax
import jax.numpy as jnp
from jax.experimental import pallas as pl

N = 10000
E = 320000
D = 128

_BN = 2000
_BE = 2000


def _gcn_pre_body(h_ref, norm_ref, wcatT_ref, bcat_ref, bias_ref,
                  hself_ref, hmn_ref):
    y = jnp.dot(h_ref[...], wcatT_ref[...], preferred_element_type=jnp.float32)
    y = y + bcat_ref[...]
    hself_ref[...] = y[:, :D] + bias_ref[...]
    hmn_ref[...] = y[:, D:] * norm_ref[...]


def _gcn_pre(h, norm, wcatT, bcat, bias):
    grid = (N // _BN,)
    return pl.pallas_call(
        _gcn_pre_body,
        grid=grid,
        in_specs=[
            pl.BlockSpec((_BN, D), lambda i: (i, 0)),
            pl.BlockSpec((_BN, 1), lambda i: (i, 0)),
            pl.BlockSpec((D, 7 * D), lambda i: (0, 0)),
            pl.BlockSpec((1, 7 * D), lambda i: (0, 0)),
            pl.BlockSpec((1, D), lambda i: (0, 0)),
        ],
        out_specs=[
            pl.BlockSpec((_BN, D), lambda i: (i, 0)),
            pl.BlockSpec((_BN, 6 * D), lambda i: (i, 0)),
        ],
        out_shape=[
            jax.ShapeDtypeStruct((N, D), jnp.float32),
            jax.ShapeDtypeStruct((N, 6 * D), jnp.float32),
        ],
    )(h, norm, wcatT, bcat.reshape(1, -1), bias)


def _msg_body(smn_ref, wdeg_ref, weT_ref, msg_ref):
    wdeg = wdeg_ref[...]
    msg = jnp.dot(wdeg, weT_ref[...], preferred_element_type=jnp.float32)
    smn = smn_ref[...]
    for i in range(6):
        gate = (wdeg[:, i:i + 1] > 0).astype(jnp.float32)
        msg = msg + smn[:, i * D:(i + 1) * D] * gate
    msg_ref[...] = msg


def _msg(smn_src, wdeg8, weT8):
    grid = (E // _BE,)
    return pl.pallas_call(
        _msg_body,
        grid=grid,
        in_specs=[
            pl.BlockSpec((_BE, 6 * D), lambda i: (i, 0)),
            pl.BlockSpec((_BE, 8), lambda i: (i, 0)),
            pl.BlockSpec((8, D), lambda i: (0, 0)),
        ],
        out_specs=pl.BlockSpec((_BE, D), lambda i: (i, 0)),
        out_shape=jax.ShapeDtypeStruct((E, D), jnp.float32),
    )(smn_src, wdeg8, weT8)


def _post_body(acc_ref, norm_ref, hself_ref, out_ref):
    out_ref[...] = jnp.maximum(
        acc_ref[...] * norm_ref[...] + hself_ref[...], 0.0)


def _post(accum, norm, hself_b):
    grid = (N // _BN,)
    return pl.pallas_call(
        _post_body,
        grid=grid,
        in_specs=[
            pl.BlockSpec((_BN, D), lambda i: (i, 0)),
            pl.BlockSpec((_BN, 1), lambda i: (i, 0)),
            pl.BlockSpec((_BN, D), lambda i: (i, 0)),
        ],
        out_specs=pl.BlockSpec((_BN, D), lambda i: (i, 0)),
        out_shape=jax.ShapeDtypeStruct((N, D), jnp.float32),
    )(accum, norm, hself_b)


def _lstm_body(x_ref, h_ref, c_ref, wiT_ref, whT_ref, b_ref, h2_ref, c2_ref):
    g = (jnp.dot(x_ref[...], wiT_ref[...], preferred_element_type=jnp.float32)
         + jnp.dot(h_ref[...], whT_ref[...], preferred_element_type=jnp.float32)
         + b_ref[...])
    i = jax.nn.sigmoid(g[:, 0 * D:1 * D])
    f = jax.nn.sigmoid(g[:, 1 * D:2 * D])
    gg = jnp.tanh(g[:, 2 * D:3 * D])
    o = jax.nn.sigmoid(g[:, 3 * D:4 * D])
    c2 = f * c_ref[...] + i * gg
    h2_ref[...] = o * jnp.tanh(c2)
    c2_ref[...] = c2


def _lstm(x, h0, c0, p):
    wiT = p['W_ih'].T
    whT = p['W_hh'].T
    b = (p['b_ih'] + p['b_hh']).reshape(1, -1)
    grid = (N // _BN,)
    return pl.pallas_call(
        _lstm_body,
        grid=grid,
        in_specs=[
            pl.BlockSpec((_BN, D), lambda i: (i, 0)),
            pl.BlockSpec((_BN, D), lambda i: (i, 0)),
            pl.BlockSpec((_BN, D), lambda i: (i, 0)),
            pl.BlockSpec((D, 4 * D), lambda i: (0, 0)),
            pl.BlockSpec((D, 4 * D), lambda i: (0, 0)),
            pl.BlockSpec((1, 4 * D), lambda i: (0, 0)),
        ],
        out_specs=[
            pl.BlockSpec((_BN, D), lambda i: (i, 0)),
            pl.BlockSpec((_BN, D), lambda i: (i, 0)),
        ],
        out_shape=[
            jax.ShapeDtypeStruct((N, D), jnp.float32),
            jax.ShapeDtypeStruct((N, D), jnp.float32),
        ],
    )(x, h0, c0, wiT, whT, b)


def _gru_body(x_ref, h_ref, wiT_ref, whT_ref, bi_ref, bh_ref, out_ref):
    gi = (jnp.dot(x_ref[...], wiT_ref[...], preferred_element_type=jnp.float32)
          + bi_ref[...])
    gh = (jnp.dot(h_ref[...], whT_ref[...], preferred_element_type=jnp.float32)
          + bh_ref[...])
    r = jax.nn.sigmoid(gi[:, 0 * D:1 * D] + gh[:, 0 * D:1 * D])
    z = jax.nn.sigmoid(gi[:, 1 * D:2 * D] + gh[:, 1 * D:2 * D])
    n = jnp.tanh(gi[:, 2 * D:3 * D] + r * gh[:, 2 * D:3 * D])
    out_ref[...] = (1.0 - z) * n + z * h_ref[...]


def _gru(x, h, p):
    grid = (N // _BN,)
    return pl.pallas_call(
        _gru_body,
        grid=grid,
        in_specs=[
            pl.BlockSpec((_BN, D), lambda i: (i, 0)),
            pl.BlockSpec((_BN, D), lambda i: (i, 0)),
            pl.BlockSpec((D, 3 * D), lambda i: (0, 0)),
            pl.BlockSpec((D, 3 * D), lambda i: (0, 0)),
            pl.BlockSpec((1, 3 * D), lambda i: (0, 0)),
            pl.BlockSpec((1, 3 * D), lambda i: (0, 0)),
        ],
        out_specs=pl.BlockSpec((_BN, D), lambda i: (i, 0)),
        out_shape=jax.ShapeDtypeStruct((N, D), jnp.float32),
    )(x, h, p['W_ih'].T, p['W_hh'].T,
      p['b_ih'].reshape(1, -1), p['b_hh'].reshape(1, -1))


def _gcn_layer(h, p, src, dst, wdeg8, weT8, norm):
    wcatT = jnp.concatenate([p['W_self'], p['W_node']], axis=0).T
    bcat = jnp.concatenate([p['b_self'], p['b_node']], axis=0)
    hself_b, hmn = _gcn_pre(h, norm, wcatT, bcat, p['bias'])
    smn_src = hmn[src]
    msg = _msg(smn_src, wdeg8, weT8)
    accum = jax.ops.segment_sum(msg, dst, num_segments=N)
    return _post(accum, norm, hself_b)


def kernel(features, hidden_gru, hidden_states_h, hidden_states_c,
           degrees, norm, edge_index, params):
    src, dst = edge_index[0], edge_index[1]
    wdeg = norm[src] * degrees
    wdeg8 = jnp.pad(wdeg, ((0, 0), (0, 2)))
    h = features
    for p in params['pre']:
        weT8 = jnp.pad(p['W_edge'].T, ((0, 2), (0, 0)))
        h = _gcn_layer(h, p, src, dst, wdeg8, weT8, norm)
    h_lstm, c_lstm = _lstm(h, hidden_states_h, hidden_states_c, params['lstm'])
    h = h_lstm
    for p in params['post']:
        weT8 = jnp.pad(p['W_edge'].T, ((0, 2), (0, 0)))
        h = _gcn_layer(h, p, src, dst, wdeg8, weT8, norm)
    hidden_gru_out = _gru(h, hidden_gru, params['gru'])
    return (hidden_gru_out, (h_lstm, c_lstm))


# speedup vs baseline: 2.2536x; 2.0156x over previous
import functools

import jax
import jax.numpy as jnp
from jax import lax
from jax.experimental import pallas as pl
from jax.experimental.pallas import tpu as pltpu
from jax.experimental.pallas import tpu_sc as plsc

N = 10000
E = 320000
D = 128

_BN = 2000
NPAD = 10240
DUMMY = N
_CH = 128
_TILES = 32
_EPT = NPAD * _CH // _TILES
EPAD = 327680
_NCHUNK = EPAD // (_TILES * _CH)
_EPT = _NCHUNK * _CH
_SLAB = NPAD // 16


def _edge_pass(snorm, hmnneg, srcp, dstp, cdst, flags):
    mesh = plsc.VectorSubcoreMesh(core_axis_name="c", subcore_axis_name="s")

    @functools.partial(
        pl.kernel,
        out_type=jax.ShapeDtypeStruct((2, NPAD, D), jnp.float32),
        mesh=mesh,
        scratch_types=[
            pltpu.VMEM_SHARED((NPAD, D), jnp.float32),
            pltpu.VMEM((_CH,), jnp.int32),
            pltpu.VMEM((_CH,), jnp.int32),
            pltpu.VMEM((_CH,), jnp.int32),
            pltpu.VMEM((_CH,), jnp.int32),
            pltpu.VMEM((_CH, D), jnp.float32),
            pltpu.VMEM((_CH, D), jnp.float32),
            pltpu.VMEM((128,), jnp.int32),
            pltpu.SemaphoreType.DMA,
        ],
    )
    def k(snorm_h, hmnneg_h, srcp_h, dstp_h, cdst_h, flags_h, out_h,
          a_sp, src_v, dst_v, cdst_v, cidx_v, rows_v, crows_v, flags_v,
          sem):
        c = lax.axis_index("c")
        s = lax.axis_index("s")
        tile = c * 16 + s

        def zbody(i, _):
            r = i // 8
            col = (i % 8) * 16
            rows_v[r, pl.ds(col, 16)] = jnp.zeros((16,), jnp.float32)
            return 0
        lax.fori_loop(0, 128 * 8, zbody, 0)
        for b in range(_SLAB // 128):
            pltpu.sync_copy(rows_v, a_sp.at[pl.ds(s * _SLAB + b * 128, 128)])
        plsc.subcore_barrier()

        pltpu.sync_copy(flags_h.at[tile], flags_v)

        def chunk_body(j, _):
            base = tile * _EPT + j * _CH
            pltpu.sync_copy(srcp_h.at[pl.ds(base, _CH)], src_v)
            pltpu.sync_copy(dstp_h.at[pl.ds(base, _CH)], dst_v)
            pltpu.async_copy(snorm_h.at[src_v], rows_v, sem).wait()
            pltpu.sync_copy(rows_v, a_sp.at[dst_v], add=True)

            flag = flags_v[pl.ds(j, 16)][0]

            @pl.when(flag != 0)
            def _corrections():
                for i in range(6):
                    pltpu.sync_copy(cdst_h.at[i, pl.ds(base, _CH)], cdst_v)

                    def gbody(g, _):
                        sv = src_v[pl.ds(g * 16, 16)]
                        cidx_v[pl.ds(g * 16, 16)] = sv * 6 + i
                        return 0
                    lax.fori_loop(0, _CH // 16, gbody, 0)
                    pltpu.async_copy(hmnneg_h.at[cidx_v], crows_v, sem).wait()
                    pltpu.sync_copy(crows_v, a_sp.at[cdst_v], add=True)
            return 0
        lax.fori_loop(0, _NCHUNK, chunk_body, 0)

        plsc.subcore_barrier()
        for b in range(_SLAB // 128):
            start = s * _SLAB + b * 128
            pltpu.sync_copy(a_sp.at[pl.ds(start, 128)],
                            out_h.at[c, pl.ds(start, 128)])

    return k(snorm, hmnneg, srcp, dstp, cdst, flags)


def _gcn_pre_body(h_ref, norm_ref, wcatT_ref, bcat_ref, bias_ref,
                  hself_ref, snorm_ref, hmnneg_ref):
    y = jnp.dot(h_ref[...], wcatT_ref[...], preferred_element_type=jnp.float32)
    y = y + bcat_ref[...]
    norm = norm_ref[...]
    hself_ref[...] = y[:, :D] + bias_ref[...]
    hm = y[:, D:]
    s = hm[:, :D]
    for i in range(1, 6):
        s = s + hm[:, i * D:(i + 1) * D]
    snorm_ref[...] = s * norm
    hmnneg_ref[...] = hm * (-norm)


def _gcn_pre(h, norm, wcatT, bcat, bias):
    grid = (N // _BN,)
    return pl.pallas_call(
        _gcn_pre_body,
        grid=grid,
        in_specs=[
            pl.BlockSpec((_BN, D), lambda i: (i, 0)),
            pl.BlockSpec((_BN, 1), lambda i: (i, 0)),
            pl.BlockSpec((D, 7 * D), lambda i: (0, 0)),
            pl.BlockSpec((1, 7 * D), lambda i: (0, 0)),
            pl.BlockSpec((1, D), lambda i: (0, 0)),
        ],
        out_specs=[
            pl.BlockSpec((_BN, D), lambda i: (i, 0)),
            pl.BlockSpec((_BN, D), lambda i: (i, 0)),
            pl.BlockSpec((_BN, 6 * D), lambda i: (i, 0)),
        ],
        out_shape=[
            jax.ShapeDtypeStruct((N, D), jnp.float32),
            jax.ShapeDtypeStruct((N, D), jnp.float32),
            jax.ShapeDtypeStruct((N, 6 * D), jnp.float32),
        ],
    )(h, norm, wcatT, bcat.reshape(1, -1), bias)


def _post_body(a0_ref, a1_ref, dseg_ref, weT_ref, norm_ref, hself_ref,
               out_ref):
    acc = (a0_ref[...] + a1_ref[...]
           + jnp.dot(dseg_ref[...], weT_ref[...],
                     preferred_element_type=jnp.float32))
    out_ref[...] = jnp.maximum(acc * norm_ref[...] + hself_ref[...], 0.0)


def _post(a0, a1, dseg8, weT8, norm, hself_b):
    grid = (N // _BN,)
    return pl.pallas_call(
        _post_body,
        grid=grid,
        in_specs=[
            pl.BlockSpec((_BN, D), lambda i: (i, 0)),
            pl.BlockSpec((_BN, D), lambda i: (i, 0)),
            pl.BlockSpec((_BN, 8), lambda i: (i, 0)),
            pl.BlockSpec((8, D), lambda i: (0, 0)),
            pl.BlockSpec((_BN, 1), lambda i: (i, 0)),
            pl.BlockSpec((_BN, D), lambda i: (i, 0)),
        ],
        out_specs=pl.BlockSpec((_BN, D), lambda i: (i, 0)),
        out_shape=jax.ShapeDtypeStruct((N, D), jnp.float32),
    )(a0, a1, dseg8, weT8, norm, hself_b)


def _lstm_body(x_ref, h_ref, c_ref, wiT_ref, whT_ref, bi_ref, bh_ref,
               h2_ref, c2_ref):
    g = (jnp.dot(x_ref[...], wiT_ref[...], preferred_element_type=jnp.float32)
         + bi_ref[...]
         + jnp.dot(h_ref[...], whT_ref[...], preferred_element_type=jnp.float32)
         + bh_ref[...])
    i = jax.nn.sigmoid(g[:, 0 * D:1 * D])
    f = jax.nn.sigmoid(g[:, 1 * D:2 * D])
    gg = jnp.tanh(g[:, 2 * D:3 * D])
    o = jax.nn.sigmoid(g[:, 3 * D:4 * D])
    c2 = f * c_ref[...] + i * gg
    h2_ref[...] = o * jnp.tanh(c2)
    c2_ref[...] = c2


def _lstm(x, h0, c0, p):
    grid = (N // _BN,)
    return pl.pallas_call(
        _lstm_body,
        grid=grid,
        in_specs=[
            pl.BlockSpec((_BN, D), lambda i: (i, 0)),
            pl.BlockSpec((_BN, D), lambda i: (i, 0)),
            pl.BlockSpec((_BN, D), lambda i: (i, 0)),
            pl.BlockSpec((D, 4 * D), lambda i: (0, 0)),
            pl.BlockSpec((D, 4 * D), lambda i: (0, 0)),
            pl.BlockSpec((1, 4 * D), lambda i: (0, 0)),
            pl.BlockSpec((1, 4 * D), lambda i: (0, 0)),
        ],
        out_specs=[
            pl.BlockSpec((_BN, D), lambda i: (i, 0)),
            pl.BlockSpec((_BN, D), lambda i: (i, 0)),
        ],
        out_shape=[
            jax.ShapeDtypeStruct((N, D), jnp.float32),
            jax.ShapeDtypeStruct((N, D), jnp.float32),
        ],
    )(x, h0, c0, p['W_ih'].T, p['W_hh'].T,
      p['b_ih'].reshape(1, -1), p['b_hh'].reshape(1, -1))


def _gru_body(x_ref, h_ref, wiT_ref, whT_ref, bi_ref, bh_ref, out_ref):
    gi = (jnp.dot(x_ref[...], wiT_ref[...], preferred_element_type=jnp.float32)
          + bi_ref[...])
    gh = (jnp.dot(h_ref[...], whT_ref[...], preferred_element_type=jnp.float32)
          + bh_ref[...])
    r = jax.nn.sigmoid(gi[:, 0 * D:1 * D] + gh[:, 0 * D:1 * D])
    z = jax.nn.sigmoid(gi[:, 1 * D:2 * D] + gh[:, 1 * D:2 * D])
    n = jnp.tanh(gi[:, 2 * D:3 * D] + r * gh[:, 2 * D:3 * D])
    out_ref[...] = (1.0 - z) * n + z * h_ref[...]


def _gru(x, h, p):
    grid = (N // _BN,)
    return pl.pallas_call(
        _gru_body,
        grid=grid,
        in_specs=[
            pl.BlockSpec((_BN, D), lambda i: (i, 0)),
            pl.BlockSpec((_BN, D), lambda i: (i, 0)),
            pl.BlockSpec((D, 3 * D), lambda i: (0, 0)),
            pl.BlockSpec((D, 3 * D), lambda i: (0, 0)),
            pl.BlockSpec((1, 3 * D), lambda i: (0, 0)),
            pl.BlockSpec((1, 3 * D), lambda i: (0, 0)),
        ],
        out_specs=pl.BlockSpec((_BN, D), lambda i: (i, 0)),
        out_shape=jax.ShapeDtypeStruct((N, D), jnp.float32),
    )(x, h, p['W_ih'].T, p['W_hh'].T,
      p['b_ih'].reshape(1, -1), p['b_hh'].reshape(1, -1))


def _gcn_layer(h, p, srcp, dstp, cdst, flags, dseg8, norm):
    wcatT = jnp.concatenate([p['W_self'], p['W_node']], axis=0).T
    bcat = jnp.concatenate([p['b_self'], p['b_node']], axis=0)
    hself_b, snorm, hmnneg = _gcn_pre(h, norm, wcatT, bcat, p['bias'])
    apart = _edge_pass(snorm, hmnneg.reshape(N * 6, D), srcp, dstp, cdst,
                       flags)
    weT8 = jnp.pad(p['W_edge'].T, ((0, 2), (0, 0)))
    return _post(apart[0, :N], apart[1, :N], dseg8, weT8, norm, hself_b)


def kernel(features, hidden_gru, hidden_states_h, hidden_states_c,
           degrees, norm, edge_index, params):
    src = edge_index[0].astype(jnp.int32)
    dst = edge_index[1].astype(jnp.int32)

    wdeg = norm[src] * degrees
    dseg = jax.ops.segment_sum(wdeg, dst, num_segments=N)
    dseg8 = jnp.pad(dseg, ((0, 0), (0, 2)))
    z = degrees == 0.0
    cdst = jnp.where(z.T, dst[None, :], DUMMY)
    cdst = jnp.pad(cdst, ((0, 0), (0, EPAD - E)),
                   constant_values=DUMMY).astype(jnp.int32)
    srcp = jnp.pad(src, (0, EPAD - E))
    dstp = jnp.pad(dst, (0, EPAD - E), constant_values=DUMMY)
    anyz = jnp.pad(jnp.any(z, axis=1), (0, EPAD - E))
    flags = jnp.any(anyz.reshape(_TILES, _NCHUNK, _CH),
                    axis=-1).astype(jnp.int32)
    flags = jnp.pad(flags, ((0, 0), (0, 128 - _NCHUNK)))

    h = features
    for p in params['pre']:
        h = _gcn_layer(h, p, srcp, dstp, cdst, flags, dseg8, norm)
    h_lstm, c_lstm = _lstm(h, hidden_states_h, hidden_states_c, params['lstm'])
    h = h_lstm
    for p in params['post']:
        h = _gcn_layer(h, p, srcp, dstp, cdst, flags, dseg8, norm)
    hidden_gru_out = _gru(h, hidden_gru, params['gru'])
    return (hidden_gru_out, (h_lstm, c_lstm))


# speedup vs baseline: 2.3000x; 1.0206x over previous
import functools

import jax
import jax.numpy as jnp
from jax import lax
from jax.experimental import pallas as pl
from jax.experimental.pallas import tpu as pltpu
from jax.experimental.pallas import tpu_sc as plsc

N = 10000
E = 320000
D = 128

_BN = 2000
NPAD = 10240
DUMMY = N
_CH = 128
_TILES = 32
_EPT = NPAD * _CH // _TILES
EPAD = 327680
_NCHUNK = EPAD // (_TILES * _CH)
_EPT = _NCHUNK * _CH
_SLAB = NPAD // 16


def _edge_pass(snorm, hmnneg, srcp, dstp, cdst, flags):
    mesh = plsc.VectorSubcoreMesh(core_axis_name="c", subcore_axis_name="s")

    @functools.partial(
        pl.kernel,
        out_type=jax.ShapeDtypeStruct((2, NPAD, D), jnp.float32),
        mesh=mesh,
        scratch_types=[
            pltpu.VMEM_SHARED((NPAD, D), jnp.float32),
            pltpu.VMEM((_CH,), jnp.int32),
            pltpu.VMEM((_CH,), jnp.int32),
            pltpu.VMEM((_CH,), jnp.int32),
            pltpu.VMEM((_CH,), jnp.int32),
            pltpu.VMEM((_CH, D), jnp.float32),
            pltpu.VMEM((_CH, D), jnp.float32),
            pltpu.VMEM((128,), jnp.int32),
            pltpu.SemaphoreType.DMA,
        ],
    )
    def k(snorm_h, hmnneg_h, srcp_h, dstp_h, cdst_h, flags_h, out_h,
          a_sp, src_v, dst_v, cdst_v, cidx_v, rows_v, crows_v, flags_v,
          sem):
        c = lax.axis_index("c")
        s = lax.axis_index("s")
        tile = c * 16 + s

        def zbody(i, _):
            r = i // 8
            col = (i % 8) * 16
            rows_v[r, pl.ds(col, 16)] = jnp.zeros((16,), jnp.float32)
            return 0
        lax.fori_loop(0, 128 * 8, zbody, 0)
        for b in range(_SLAB // 128):
            pltpu.sync_copy(rows_v, a_sp.at[pl.ds(s * _SLAB + b * 128, 128)])
        plsc.subcore_barrier()

        pltpu.sync_copy(flags_h.at[tile], flags_v)

        def chunk_body(j, _):
            base = tile * _EPT + j * _CH
            pltpu.sync_copy(srcp_h.at[pl.ds(base, _CH)], src_v)
            pltpu.sync_copy(dstp_h.at[pl.ds(base, _CH)], dst_v)
            pltpu.async_copy(snorm_h.at[src_v], rows_v, sem).wait()
            pltpu.sync_copy(rows_v, a_sp.at[dst_v], add=True)

            flag = flags_v[pl.ds(j, 16)][0]

            @pl.when(flag != 0)
            def _corrections():
                for i in range(6):
                    pltpu.sync_copy(cdst_h.at[i, pl.ds(base, _CH)], cdst_v)

                    def gbody(g, _):
                        sv = src_v[pl.ds(g * 16, 16)]
                        cidx_v[pl.ds(g * 16, 16)] = sv * 6 + i
                        return 0
                    lax.fori_loop(0, _CH // 16, gbody, 0)
                    pltpu.async_copy(hmnneg_h.at[cidx_v], crows_v, sem).wait()
                    pltpu.sync_copy(crows_v, a_sp.at[cdst_v], add=True)
            return 0
        lax.fori_loop(0, _NCHUNK, chunk_body, 0)

        plsc.subcore_barrier()
        for b in range(_SLAB // 128):
            start = s * _SLAB + b * 128
            pltpu.sync_copy(a_sp.at[pl.ds(start, 128)],
                            out_h.at[c, pl.ds(start, 128)])

    return k(snorm, hmnneg, srcp, dstp, cdst, flags)


def _gcn_pre_body(h_ref, norm_ref, wcatT_ref, bcat_ref, bias_ref,
                  hself_ref, snorm_ref, hmnneg_ref):
    y = jnp.dot(h_ref[...], wcatT_ref[...], preferred_element_type=jnp.float32)
    y = y + bcat_ref[...]
    norm = norm_ref[...]
    hself_ref[...] = y[:, :D] + bias_ref[...]
    hm = y[:, D:]
    s = hm[:, :D]
    for i in range(1, 6):
        s = s + hm[:, i * D:(i + 1) * D]
    snorm_ref[...] = s * norm
    hmnneg_ref[...] = hm * (-norm)


def _gcn_pre(h, norm, wcatT, bcat, bias):
    grid = (N // _BN,)
    return pl.pallas_call(
        _gcn_pre_body,
        grid=grid,
        in_specs=[
            pl.BlockSpec((_BN, D), lambda i: (i, 0)),
            pl.BlockSpec((_BN, 1), lambda i: (i, 0)),
            pl.BlockSpec((D, 7 * D), lambda i: (0, 0)),
            pl.BlockSpec((1, 7 * D), lambda i: (0, 0)),
            pl.BlockSpec((1, D), lambda i: (0, 0)),
        ],
        out_specs=[
            pl.BlockSpec((_BN, D), lambda i: (i, 0)),
            pl.BlockSpec((_BN, D), lambda i: (i, 0)),
            pl.BlockSpec((_BN, 6 * D), lambda i: (i, 0)),
        ],
        out_shape=[
            jax.ShapeDtypeStruct((N, D), jnp.float32),
            jax.ShapeDtypeStruct((N, D), jnp.float32),
            jax.ShapeDtypeStruct((N, 6 * D), jnp.float32),
        ],
    )(h, norm, wcatT, bcat.reshape(1, -1), bias)


def _post_body(a0_ref, a1_ref, dseg_ref, weT_ref, norm_ref, hself_ref,
               out_ref):
    acc = (a0_ref[...] + a1_ref[...]
           + jnp.dot(dseg_ref[...], weT_ref[...],
                     preferred_element_type=jnp.float32))
    out_ref[...] = jnp.maximum(acc * norm_ref[...] + hself_ref[...], 0.0)


def _post(a0, a1, dseg8, weT8, norm, hself_b):
    grid = (N // _BN,)
    return pl.pallas_call(
        _post_body,
        grid=grid,
        in_specs=[
            pl.BlockSpec((_BN, D), lambda i: (i, 0)),
            pl.BlockSpec((_BN, D), lambda i: (i, 0)),
            pl.BlockSpec((_BN, 8), lambda i: (i, 0)),
            pl.BlockSpec((8, D), lambda i: (0, 0)),
            pl.BlockSpec((_BN, 1), lambda i: (i, 0)),
            pl.BlockSpec((_BN, D), lambda i: (i, 0)),
        ],
        out_specs=pl.BlockSpec((_BN, D), lambda i: (i, 0)),
        out_shape=jax.ShapeDtypeStruct((N, D), jnp.float32),
    )(a0, a1, dseg8, weT8, norm, hself_b)


def _lstm_body(x_ref, h_ref, c_ref, wiT_ref, whT_ref, bi_ref, bh_ref,
               h2_ref, c2_ref):
    g = (jnp.dot(x_ref[...], wiT_ref[...], preferred_element_type=jnp.float32)
         + bi_ref[...]
         + jnp.dot(h_ref[...], whT_ref[...], preferred_element_type=jnp.float32)
         + bh_ref[...])
    i = jax.nn.sigmoid(g[:, 0 * D:1 * D])
    f = jax.nn.sigmoid(g[:, 1 * D:2 * D])
    gg = jnp.tanh(g[:, 2 * D:3 * D])
    o = jax.nn.sigmoid(g[:, 3 * D:4 * D])
    c2 = f * c_ref[...] + i * gg
    h2_ref[...] = o * jnp.tanh(c2)
    c2_ref[...] = c2


def _lstm(x, h0, c0, p):
    grid = (N // _BN,)
    return pl.pallas_call(
        _lstm_body,
        grid=grid,
        in_specs=[
            pl.BlockSpec((_BN, D), lambda i: (i, 0)),
            pl.BlockSpec((_BN, D), lambda i: (i, 0)),
            pl.BlockSpec((_BN, D), lambda i: (i, 0)),
            pl.BlockSpec((D, 4 * D), lambda i: (0, 0)),
            pl.BlockSpec((D, 4 * D), lambda i: (0, 0)),
            pl.BlockSpec((1, 4 * D), lambda i: (0, 0)),
            pl.BlockSpec((1, 4 * D), lambda i: (0, 0)),
        ],
        out_specs=[
            pl.BlockSpec((_BN, D), lambda i: (i, 0)),
            pl.BlockSpec((_BN, D), lambda i: (i, 0)),
        ],
        out_shape=[
            jax.ShapeDtypeStruct((N, D), jnp.float32),
            jax.ShapeDtypeStruct((N, D), jnp.float32),
        ],
    )(x, h0, c0, p['W_ih'].T, p['W_hh'].T,
      p['b_ih'].reshape(1, -1), p['b_hh'].reshape(1, -1))


def _gru_body(x_ref, h_ref, wiT_ref, whT_ref, bi_ref, bh_ref, out_ref):
    gi = (jnp.dot(x_ref[...], wiT_ref[...], preferred_element_type=jnp.float32)
          + bi_ref[...])
    gh = (jnp.dot(h_ref[...], whT_ref[...], preferred_element_type=jnp.float32)
          + bh_ref[...])
    r = jax.nn.sigmoid(gi[:, 0 * D:1 * D] + gh[:, 0 * D:1 * D])
    z = jax.nn.sigmoid(gi[:, 1 * D:2 * D] + gh[:, 1 * D:2 * D])
    n = jnp.tanh(gi[:, 2 * D:3 * D] + r * gh[:, 2 * D:3 * D])
    out_ref[...] = (1.0 - z) * n + z * h_ref[...]


def _gru(x, h, p):
    grid = (N // _BN,)
    return pl.pallas_call(
        _gru_body,
        grid=grid,
        in_specs=[
            pl.BlockSpec((_BN, D), lambda i: (i, 0)),
            pl.BlockSpec((_BN, D), lambda i: (i, 0)),
            pl.BlockSpec((D, 3 * D), lambda i: (0, 0)),
            pl.BlockSpec((D, 3 * D), lambda i: (0, 0)),
            pl.BlockSpec((1, 3 * D), lambda i: (0, 0)),
            pl.BlockSpec((1, 3 * D), lambda i: (0, 0)),
        ],
        out_specs=pl.BlockSpec((_BN, D), lambda i: (i, 0)),
        out_shape=jax.ShapeDtypeStruct((N, D), jnp.float32),
    )(x, h, p['W_ih'].T, p['W_hh'].T,
      p['b_ih'].reshape(1, -1), p['b_hh'].reshape(1, -1))


def _gcn_layer(h, p, srcp, dstp, cdst, flags, dseg8, norm):
    wcatT = jnp.concatenate([p['W_self'], p['W_node']], axis=0).T
    bcat = jnp.concatenate([p['b_self'], p['b_node']], axis=0)
    hself_b, snorm, hmnneg = _gcn_pre(h, norm, wcatT, bcat, p['bias'])
    apart = _edge_pass(snorm, hmnneg.reshape(N * 6, D), srcp, dstp, cdst,
                       flags)
    weT8 = jnp.pad(p['W_edge'].T, ((0, 2), (0, 0)))
    return _post(apart[0, :N], apart[1, :N], dseg8, weT8, norm, hself_b)


def kernel(features, hidden_gru, hidden_states_h, hidden_states_c,
           degrees, norm, edge_index, params):
    src = edge_index[0].astype(jnp.int32)
    dst = edge_index[1].astype(jnp.int32)

    z = degrees == 0.0
    cdst = jnp.where(z.T, dst[None, :], DUMMY)
    cdst = jnp.pad(cdst, ((0, 0), (0, EPAD - E)),
                   constant_values=DUMMY).astype(jnp.int32)
    srcp = jnp.pad(src, (0, EPAD - E))
    dstp = jnp.pad(dst, (0, EPAD - E), constant_values=DUMMY)
    anyz = jnp.pad(jnp.any(z, axis=1), (0, EPAD - E))
    flags = jnp.any(anyz.reshape(_TILES, _NCHUNK, _CH),
                    axis=-1).astype(jnp.int32)
    flags = jnp.pad(flags, ((0, 0), (0, 128 - _NCHUNK)))

    wdeg = norm[src] * degrees
    dseg = jax.ops.segment_sum(wdeg, dst, num_segments=N)
    dseg8 = jnp.pad(dseg, ((0, 0), (0, 2)))

    h = features
    for p in params['pre']:
        h = _gcn_layer(h, p, srcp, dstp, cdst, flags, dseg8, norm)
    h_lstm, c_lstm = _lstm(h, hidden_states_h, hidden_states_c, params['lstm'])
    h = h_lstm
    for p in params['post']:
        h = _gcn_layer(h, p, srcp, dstp, cdst, flags, dseg8, norm)
    hidden_gru_out = _gru(h, hidden_gru, params['gru'])
    return (hidden_gru_out, (h_lstm, c_lstm))
